# Initial kernel scaffold; baseline (speedup 1.0000x reference)
import jax
import jax.numpy as jnp
from jax.experimental import pallas as pl


def kernel(pred_heatmap, pred_regression):
    raise NotImplementedError("write your pallas kernel here")


# speedup vs baseline: 1.0268x; 1.0268x over previous
import functools

import jax
import jax.numpy as jnp
from jax.experimental import pallas as pl

B, C, H, W = 8, 3, 192, 640
R = 50
K = 100
DET_THRESHOLD = 0.25
HW = H * W
CHW = C * H * W


def _nms_body(x_ref, o_ref):
    x = x_ref[0, 0]
    ninf = jnp.float32(-jnp.inf)
    ninf_row = jnp.full((1, W), ninf, dtype=jnp.float32)
    up = jnp.concatenate([x[1:, :], ninf_row], axis=0)
    down = jnp.concatenate([ninf_row, x[:-1, :]], axis=0)
    m = jnp.maximum(jnp.maximum(x, up), down)
    ninf_col = jnp.full((H, 1), ninf, dtype=jnp.float32)
    left = jnp.concatenate([m[:, 1:], ninf_col], axis=1)
    right = jnp.concatenate([ninf_col, m[:, :-1]], axis=1)
    pooled = jnp.maximum(jnp.maximum(m, left), right)
    o_ref[0, 0] = jnp.where(pooled == x, x, jnp.float32(0.0))


def _nms(heat):
    return pl.pallas_call(
        _nms_body,
        grid=(B, C),
        in_specs=[pl.BlockSpec((1, 1, H, W), lambda b, c: (b, c, 0, 0))],
        out_specs=pl.BlockSpec((1, 1, H, W), lambda b, c: (b, c, 0, 0)),
        out_shape=jax.ShapeDtypeStruct((B, C, H, W), jnp.float32),
    )(heat)


def kernel(pred_heatmap, pred_regression):
    nms = _nms(pred_heatmap)
    flat = nms.reshape(B, CHW)
    scores, idx = jax.lax.top_k(flat, K)
    inds = (idx % HW).astype(jnp.int32)
    ys = (inds // W).astype(jnp.float32)
    xs = (inds % W).astype(jnp.float32)

    reg_flat = pred_regression.reshape(B, R, HW)
    pois = jnp.take_along_axis(
        reg_flat, inds[:, None, :], axis=2
    ).transpose(0, 2, 1)

    scores_f = scores.reshape(-1)
    mask = (scores_f >= DET_THRESHOLD).astype(jnp.float32)
    out = jnp.concatenate(
        [
            (scores_f * mask)[:, None],
            pois.reshape(-1, R) * mask[:, None],
            jnp.stack([xs.reshape(-1), ys.reshape(-1)], axis=1) * mask[:, None],
        ],
        axis=1,
    )
    return out


# speedup vs baseline: 1.0485x; 1.0212x over previous
import functools

import jax
import jax.numpy as jnp
from jax import lax
from jax.experimental import pallas as pl
from jax.experimental.pallas import tpu as pltpu
from jax.experimental.pallas import tpu_sc as plsc

B, C, H, W = 8, 3, 192, 640
R = 50
K = 100
DET_THRESHOLD = 0.25
HW = H * W
CHW = C * H * W


def _nms_body(x_ref, o_ref):
    x = x_ref[0, 0]
    ninf = jnp.float32(-jnp.inf)
    ninf_row = jnp.full((1, W), ninf, dtype=jnp.float32)
    up = jnp.concatenate([x[1:, :], ninf_row], axis=0)
    down = jnp.concatenate([ninf_row, x[:-1, :]], axis=0)
    m = jnp.maximum(jnp.maximum(x, up), down)
    ninf_col = jnp.full((H, 1), ninf, dtype=jnp.float32)
    left = jnp.concatenate([m[:, 1:], ninf_col], axis=1)
    right = jnp.concatenate([ninf_col, m[:, :-1]], axis=1)
    pooled = jnp.maximum(jnp.maximum(m, left), right)
    o_ref[0, 0] = jnp.where(pooled == x, x, jnp.float32(0.0))


def _nms(heat):
    return pl.pallas_call(
        _nms_body,
        grid=(B, C),
        in_specs=[pl.BlockSpec((1, 1, H, W), lambda b, c: (b, c, 0, 0))],
        out_specs=pl.BlockSpec((1, 1, H, W), lambda b, c: (b, c, 0, 0)),
        out_shape=jax.ShapeDtypeStruct((B, C, H, W), jnp.float32),
    )(heat)


NW = 32
GPT = 1280
GPT_REAL = 1250


def _sc_gather(reg_flat, addr, maskv):
    mesh = plsc.VectorSubcoreMesh(core_axis_name="c", subcore_axis_name="s")

    @functools.partial(
        pl.kernel,
        mesh=mesh,
        out_type=jax.ShapeDtypeStruct((NW, GPT), jnp.float32),
        scratch_types=[
            pltpu.VMEM((GPT,), jnp.int32),
            pltpu.VMEM((GPT,), jnp.float32),
            pltpu.VMEM((GPT,), jnp.float32),
            pltpu.SemaphoreType.DMA,
        ],
    )
    def gather_k(reg_hbm, addr_hbm, mask_hbm, out_hbm, idx_v, rows_v, mask_v, sem):
        wid = lax.axis_index("s") * 2 + lax.axis_index("c")
        pltpu.sync_copy(addr_hbm.at[wid], idx_v)
        pltpu.sync_copy(mask_hbm.at[wid], mask_v)
        pltpu.async_copy(reg_hbm.at[idx_v], rows_v, sem).wait()
        for j in range(GPT // 16):
            sl = pl.ds(j * 16, 16)
            rows_v[sl] = rows_v[sl] * mask_v[sl]
        pltpu.sync_copy(rows_v, out_hbm.at[wid])

    return gather_k(reg_flat, addr, maskv)


def kernel(pred_heatmap, pred_regression):
    nms = _nms(pred_heatmap)
    flat = nms.reshape(B, CHW)
    scores, idx = jax.lax.top_k(flat, K)
    inds = (idx % HW).astype(jnp.int32)
    ys = (inds // W).astype(jnp.float32)
    xs = (inds % W).astype(jnp.float32)

    scores_f = scores.reshape(-1)
    mask = (scores_f >= DET_THRESHOLD).astype(jnp.float32)

    addr = (
        inds[:, :, None]
        + (jnp.arange(R, dtype=jnp.int32) * HW)[None, None, :]
        + (jnp.arange(B, dtype=jnp.int32) * (R * HW))[:, None, None]
    ).reshape(NW, GPT_REAL)
    addr = jnp.concatenate(
        [addr, jnp.zeros((NW, GPT - GPT_REAL), jnp.int32)], axis=1
    )
    maskv = jnp.broadcast_to(mask.reshape(B, K, 1), (B, K, R)).reshape(NW, GPT_REAL)
    maskv = jnp.concatenate(
        [maskv, jnp.zeros((NW, GPT - GPT_REAL), jnp.float32)], axis=1
    )
    pois_m = _sc_gather(pred_regression.reshape(-1), addr, maskv)
    pois_m = pois_m[:, :GPT_REAL].reshape(B * K, R)

    out = jnp.concatenate(
        [
            (scores_f * mask)[:, None],
            pois_m,
            jnp.stack([xs.reshape(-1), ys.reshape(-1)], axis=1) * mask[:, None],
        ],
        axis=1,
    )
    return out


# speedup vs baseline: 2.9932x; 2.8546x over previous
import functools

import jax
import jax.numpy as jnp
from jax import lax
from jax.experimental import pallas as pl
from jax.experimental.pallas import tpu as pltpu
from jax.experimental.pallas import tpu_sc as plsc

B, C, H, W = 8, 3, 192, 640
R = 50
K = 100
DET_THRESHOLD = 0.25
HW = H * W
CHW = C * H * W


def _nms_body(x_ref, o_ref):
    x = x_ref[0, 0]
    ninf = jnp.float32(-jnp.inf)
    ninf_row = jnp.full((1, W), ninf, dtype=jnp.float32)
    up = jnp.concatenate([x[1:, :], ninf_row], axis=0)
    down = jnp.concatenate([ninf_row, x[:-1, :]], axis=0)
    m = jnp.maximum(jnp.maximum(x, up), down)
    ninf_col = jnp.full((H, 1), ninf, dtype=jnp.float32)
    left = jnp.concatenate([m[:, 1:], ninf_col], axis=1)
    right = jnp.concatenate([ninf_col, m[:, :-1]], axis=1)
    pooled = jnp.maximum(jnp.maximum(m, left), right)
    o_ref[0, 0] = jnp.where(pooled == x, x, jnp.float32(0.0))


def _nms(heat):
    return pl.pallas_call(
        _nms_body,
        grid=(B, C),
        in_specs=[pl.BlockSpec((1, 1, H, W), lambda b, c: (b, c, 0, 0))],
        out_specs=pl.BlockSpec((1, 1, H, W), lambda b, c: (b, c, 0, 0)),
        out_shape=jax.ShapeDtypeStruct((B, C, H, W), jnp.float32),
    )(heat)


NB = 512
CHUNK = CHW // 16
NVEC = CHUNK // 16
CCAP = 128
PCAP = 1024
KPAD = 128


def _sc_topk(nms_flat):
    mesh = plsc.VectorSubcoreMesh(core_axis_name="c", subcore_axis_name="s")
    i32 = jnp.int32
    f32 = jnp.float32

    @functools.partial(
        pl.kernel,
        mesh=mesh,
        compiler_params=pltpu.CompilerParams(needs_layout_passes=False),
        out_type=(
            jax.ShapeDtypeStruct((B, KPAD), f32),
            jax.ShapeDtypeStruct((B, KPAD), i32),
            jax.ShapeDtypeStruct((B, KPAD), f32),
            jax.ShapeDtypeStruct((B, KPAD), f32),
        ),
        scratch_types=[
            pltpu.VMEM((4, CHUNK), f32),
            pltpu.VMEM((16 * NB,), i32),
            pltpu.VMEM((NB,), i32),
            pltpu.VMEM((16,), f32),
            pltpu.VMEM((4, 16), f32),
            pltpu.VMEM((4, CCAP), f32),
            pltpu.VMEM((4, CCAP), i32),
            pltpu.VMEM((16 * CCAP,), f32),
            pltpu.VMEM((16 * CCAP,), i32),
            pltpu.VMEM((PCAP,), f32),
            pltpu.VMEM((PCAP,), i32),
            pltpu.VMEM((KPAD,), f32),
            pltpu.VMEM((KPAD,), i32),
            pltpu.VMEM((KPAD,), f32),
            pltpu.VMEM((KPAD,), f32),
            pltpu.VMEM_SHARED((4, 16, NB), i32),
            pltpu.VMEM_SHARED((4, 16), f32),
            pltpu.VMEM_SHARED((4, 16, CCAP), f32),
            pltpu.VMEM_SHARED((4, 16, CCAP), i32),
        ],
    )
    def topk_k(nms_hbm, s_hbm, i_hbm, x_hbm, y_hbm,
               chunk_v, hist2d, local_hist, thresh_v, tread_v, cscore_v, cidx_v,
               pools_s, pools_i, dense_s, dense_i, out_s, out_i, out_x, out_y,
               hist_sh, thresh_sh, pool_s_sh, pool_i_sh):
        cid = lax.axis_index("c")
        sid = lax.axis_index("s")
        lane = lax.iota(i32, 16)
        zeros16i = jnp.zeros((16,), i32)
        ones16i = jnp.ones((16,), i32)
        lane_off = lane * NB
        chunk_base = sid * CHUNK

        for bl in range(4):
            gb = cid * 4 + bl
            pltpu.sync_copy(
                nms_hbm.at[gb, pl.ds(chunk_base, CHUNK)], chunk_v.at[bl]
            )

            def zero_body(j, _):
                hist2d[pl.ds(j * 16, 16)] = zeros16i
                return 0

            lax.fori_loop(0, 16 * NB // 16, zero_body, 0)

            def hist_body(j, _):
                v = chunk_v[bl, pl.ds(j * 16, 16)]
                bins = (v * NB).astype(i32)
                plsc.addupdate_scatter(hist2d, [lane_off + bins], ones16i)
                return 0

            lax.fori_loop(0, NVEC, hist_body, 0)

            def lred_body(j, _):
                acc = zeros16i
                for l in range(16):
                    acc = acc + hist2d[pl.ds(l * NB + j * 16, 16)]
                local_hist[pl.ds(j * 16, 16)] = acc
                return 0

            lax.fori_loop(0, NB // 16, lred_body, 0)
            pltpu.sync_copy(local_hist, hist_sh.at[bl, sid])

        plsc.subcore_barrier()

        @pl.when(sid < 4)
        def _():
            bl = sid
            for t in range(16):
                pltpu.sync_copy(hist_sh.at[bl, t], hist2d.at[pl.ds(t * NB, NB)])

            def tred_body(j, _):
                acc = zeros16i
                for t in range(16):
                    acc = acc + hist2d[pl.ds(t * NB + j * 16, 16)]
                local_hist[pl.ds(j * 16, 16)] = acc
                return 0

            lax.fori_loop(0, NB // 16, tred_body, 0)

            def suf_body(jr, carry):
                j = NB // 16 - 1 - jr
                beta, csum = carry
                h = local_hist[pl.ds(j * 16, 16)]
                cs = plsc.cumsum(h)
                tot = jnp.max(cs)
                suffix = (csum + tot) - cs + h
                bins = j * 16 + lane
                cand = jnp.max(jnp.where(suffix >= K, bins, -1))
                return jnp.maximum(beta, cand), csum + tot

            beta, _ = lax.fori_loop(0, NB // 16, suf_body, (jnp.int32(-1), jnp.int32(0)))
            beta = jnp.maximum(beta, jnp.int32(NB // 4))
            tval = beta.astype(f32) * (1.0 / NB)
            thresh_v[...] = jnp.broadcast_to(tval, (16,))
            pltpu.sync_copy(thresh_v, thresh_sh.at[bl])

        plsc.subcore_barrier()

        for bl in range(4):
            for j in range(CCAP // 16):
                cscore_v[bl, pl.ds(j * 16, 16)] = jnp.full((16,), -1.0, f32)
                cidx_v[bl, pl.ds(j * 16, 16)] = zeros16i
        pltpu.sync_copy(thresh_sh, tread_v)
        for bl in range(4):
            tv = tread_v[bl, pl.ds(0, 16)]

            def comp_body(j, cntv):
                v = chunk_v[bl, pl.ds(j * 16, 16)]
                m = v >= tv
                cs = plsc.cumsum(m.astype(i32))
                pos = cntv + cs - 1
                mw = m & (pos < CCAP)
                plsc.store_scatter(cscore_v.at[bl], [pos], v, mask=mw)
                idxs = chunk_base + j * 16 + lane
                plsc.store_scatter(cidx_v.at[bl], [pos], idxs, mask=mw)
                return cntv + plsc.all_reduce_population_count(m)

            lax.fori_loop(0, NVEC, comp_body, jnp.zeros((16,), i32))
        for bl in range(4):
            for j in range(CCAP // 16):
                sl = pl.ds(j * 16, 16)
                cscore_v[bl, sl] = cscore_v[bl, sl] * jnp.float32(1.0)
                cidx_v[bl, sl] = cidx_v[bl, sl] + 0
            pltpu.sync_copy(cscore_v.at[bl], pool_s_sh.at[bl, sid])
            pltpu.sync_copy(cidx_v.at[bl], pool_i_sh.at[bl, sid])

        plsc.subcore_barrier()

        @pl.when(sid < 4)
        def _():
            bl = sid
            gb = cid * 4 + bl
            for t in range(16):
                pltpu.sync_copy(pool_s_sh.at[bl, t], pools_s.at[pl.ds(t * CCAP, CCAP)])
                pltpu.sync_copy(pool_i_sh.at[bl, t], pools_i.at[pl.ds(t * CCAP, CCAP)])
            for j in range(PCAP // 16):
                dense_s[pl.ds(j * 16, 16)] = jnp.full((16,), -1.0, f32)
                dense_i[pl.ds(j * 16, 16)] = zeros16i

            def dcomp_body(j, cntv):
                s = pools_s[pl.ds(j * 16, 16)]
                idx = pools_i[pl.ds(j * 16, 16)]
                m = s > -0.5
                cs = plsc.cumsum(m.astype(i32))
                pos = cntv + cs - 1
                mw = m & (pos < PCAP)
                plsc.store_scatter(dense_s, [pos], s, mask=mw)
                plsc.store_scatter(dense_i, [pos], idx, mask=mw)
                return cntv + plsc.all_reduce_population_count(m)

            cntv = lax.fori_loop(
                0, 16 * CCAP // 16, dcomp_body, jnp.zeros((16,), i32))
            cnt = jnp.max(cntv)
            nv = jnp.minimum((cnt + 15) // 16, PCAP // 16)
            big = jnp.full((16,), jnp.int32(2**30), i32)

            def rank_body(r, _):
                def scan_body(q, st):
                    bs, bi, bp = st
                    s = dense_s[pl.ds(q * 16, 16)]
                    idx = dense_i[pl.ds(q * 16, 16)]
                    p = q * 16 + lane
                    better = (s > bs) | ((s == bs) & (idx < bi))
                    return (
                        jnp.where(better, s, bs),
                        jnp.where(better, idx, bi),
                        jnp.where(better, p, bp),
                    )

                init = (jnp.full((16,), -2.0, f32), big, big)
                bs, bi, bp = lax.fori_loop(0, nv, scan_body, init)
                mval = jnp.max(bs)
                tie = bs == mval
                mi_ = jnp.min(jnp.where(tie, bi, big))
                mp = jnp.min(jnp.where(tie & (bi == mi_), bp, big))
                mp = jnp.minimum(mp, jnp.int32(PCAP - 1))
                lane0 = lane == 0
                plsc.store_scatter(
                    dense_s, [jnp.broadcast_to(mp, (16,))],
                    jnp.full((16,), -1.0, f32), mask=lane0)
                plsc.store_scatter(
                    out_s, [jnp.broadcast_to(r, (16,))],
                    jnp.broadcast_to(mval, (16,)), mask=lane0)
                plsc.store_scatter(
                    out_i, [jnp.broadcast_to(r, (16,))],
                    jnp.broadcast_to(mi_, (16,)), mask=lane0)
                return 0

            lax.fori_loop(0, K, rank_body, 0)

            for j in range(KPAD // 16):
                sl = pl.ds(j * 16, 16)
                s = out_s[sl]
                idx = jnp.where(s >= DET_THRESHOLD, out_i[sl] % HW, zeros16i)
                m = (s >= DET_THRESHOLD).astype(f32)
                out_s[sl] = s * m
                out_i[sl] = idx
                out_x[sl] = (idx % W).astype(f32)
                out_y[sl] = (idx // W).astype(f32)
            pltpu.sync_copy(out_s, s_hbm.at[gb])
            pltpu.sync_copy(out_i, i_hbm.at[gb])
            pltpu.sync_copy(out_x, x_hbm.at[gb])
            pltpu.sync_copy(out_y, y_hbm.at[gb])

    return topk_k(nms_flat)


NW = 32
GPT = 1280
GPT_REAL = 1250


def _sc_gather(reg_flat, addr, maskv):
    mesh = plsc.VectorSubcoreMesh(core_axis_name="c", subcore_axis_name="s")

    @functools.partial(
        pl.kernel,
        mesh=mesh,
        out_type=jax.ShapeDtypeStruct((NW, GPT), jnp.float32),
        scratch_types=[
            pltpu.VMEM((GPT,), jnp.int32),
            pltpu.VMEM((GPT,), jnp.float32),
            pltpu.VMEM((GPT,), jnp.float32),
            pltpu.SemaphoreType.DMA,
        ],
    )
    def gather_k(reg_hbm, addr_hbm, mask_hbm, out_hbm, idx_v, rows_v, mask_v, sem):
        wid = lax.axis_index("s") * 2 + lax.axis_index("c")
        pltpu.sync_copy(addr_hbm.at[wid], idx_v)
        pltpu.sync_copy(mask_hbm.at[wid], mask_v)
        pltpu.async_copy(reg_hbm.at[idx_v], rows_v, sem).wait()
        for j in range(GPT // 16):
            sl = pl.ds(j * 16, 16)
            rows_v[sl] = rows_v[sl] * mask_v[sl]
        pltpu.sync_copy(rows_v, out_hbm.at[wid])

    return gather_k(reg_flat, addr, maskv)


def kernel(pred_heatmap, pred_regression):
    nms = _nms(pred_heatmap)
    flat = nms.reshape(B, CHW)
    scores_m, idx_sp, xs_m, ys_m = _sc_topk(flat)
    scores_m = scores_m[:, :K]
    inds = idx_sp[:, :K]

    scores_f = scores_m.reshape(-1)
    mask = (scores_f >= DET_THRESHOLD).astype(jnp.float32)

    addr = (
        inds[:, :, None]
        + (jnp.arange(R, dtype=jnp.int32) * HW)[None, None, :]
        + (jnp.arange(B, dtype=jnp.int32) * (R * HW))[:, None, None]
    ).reshape(NW, GPT_REAL)
    addr = jnp.concatenate(
        [addr, jnp.zeros((NW, GPT - GPT_REAL), jnp.int32)], axis=1
    )
    maskv = jnp.broadcast_to(mask.reshape(B, K, 1), (B, K, R)).reshape(NW, GPT_REAL)
    maskv = jnp.concatenate(
        [maskv, jnp.zeros((NW, GPT - GPT_REAL), jnp.float32)], axis=1
    )
    pois_m = _sc_gather(pred_regression.reshape(-1), addr, maskv)
    pois_m = pois_m[:, :GPT_REAL].reshape(B * K, R)

    out = jnp.concatenate(
        [
            scores_f[:, None],
            pois_m,
            jnp.stack(
                [xs_m[:, :K].reshape(-1), ys_m[:, :K].reshape(-1)], axis=1
            ),
        ],
        axis=1,
    )
    return out


# speedup vs baseline: 3.0636x; 1.0235x over previous
import functools

import jax
import jax.numpy as jnp
from jax import lax
from jax.experimental import pallas as pl
from jax.experimental.pallas import tpu as pltpu
from jax.experimental.pallas import tpu_sc as plsc

B, C, H, W = 8, 3, 192, 640
R = 50
K = 100
DET_THRESHOLD = 0.25
HW = H * W
CHW = C * H * W


def _nms_body(x_ref, o_ref):
    x = x_ref[0, 0]
    ninf = jnp.float32(-jnp.inf)
    ninf_row = jnp.full((1, W), ninf, dtype=jnp.float32)
    up = jnp.concatenate([x[1:, :], ninf_row], axis=0)
    down = jnp.concatenate([ninf_row, x[:-1, :]], axis=0)
    m = jnp.maximum(jnp.maximum(x, up), down)
    ninf_col = jnp.full((H, 1), ninf, dtype=jnp.float32)
    left = jnp.concatenate([m[:, 1:], ninf_col], axis=1)
    right = jnp.concatenate([ninf_col, m[:, :-1]], axis=1)
    pooled = jnp.maximum(jnp.maximum(m, left), right)
    o_ref[0, 0] = jnp.where(pooled == x, x, jnp.float32(0.0))


def _nms(heat):
    return pl.pallas_call(
        _nms_body,
        grid=(B, C),
        in_specs=[pl.BlockSpec((1, 1, H, W), lambda b, c: (b, c, 0, 0))],
        out_specs=pl.BlockSpec((1, 1, H, W), lambda b, c: (b, c, 0, 0)),
        out_shape=jax.ShapeDtypeStruct((B, C, H, W), jnp.float32),
    )(heat)


NB = 512
NBINS = 1024
FOFF = 504
FSUB = 32256 - FOFF
CHUNK = CHW // 16
NVEC = CHUNK // 16
CCAP = 128
PCAP = 1024
KPAD = 128


def _sc_topk(nms_flat):
    mesh = plsc.VectorSubcoreMesh(core_axis_name="c", subcore_axis_name="s")
    i32 = jnp.int32
    f32 = jnp.float32

    @functools.partial(
        pl.kernel,
        mesh=mesh,
        compiler_params=pltpu.CompilerParams(needs_layout_passes=False),
        out_type=(
            jax.ShapeDtypeStruct((B, KPAD), f32),
            jax.ShapeDtypeStruct((B, KPAD), i32),
            jax.ShapeDtypeStruct((B, KPAD), f32),
            jax.ShapeDtypeStruct((B, KPAD), f32),
        ),
        scratch_types=[
            pltpu.VMEM((4, CHUNK), f32),
            pltpu.VMEM((16 * NBINS,), i32),
            pltpu.VMEM((NBINS,), i32),
            pltpu.VMEM((16,), f32),
            pltpu.VMEM((4, 16), f32),
            pltpu.VMEM((4, CCAP), f32),
            pltpu.VMEM((4, CCAP), i32),
            pltpu.VMEM((16 * CCAP,), f32),
            pltpu.VMEM((16 * CCAP,), i32),
            pltpu.VMEM((PCAP,), f32),
            pltpu.VMEM((PCAP,), i32),
            pltpu.VMEM((KPAD,), f32),
            pltpu.VMEM((KPAD,), i32),
            pltpu.VMEM((KPAD,), f32),
            pltpu.VMEM((KPAD,), f32),
            pltpu.VMEM_SHARED((4, 16, NBINS), i32),
            pltpu.VMEM_SHARED((4, 16), f32),
            pltpu.VMEM_SHARED((4, 16, CCAP), f32),
            pltpu.VMEM_SHARED((4, 16, CCAP), i32),
        ],
    )
    def topk_k(nms_hbm, s_hbm, i_hbm, x_hbm, y_hbm,
               chunk_v, hist2d, local_hist, thresh_v, tread_v, cscore_v, cidx_v,
               pools_s, pools_i, dense_s, dense_i, out_s, out_i, out_x, out_y,
               hist_sh, thresh_sh, pool_s_sh, pool_i_sh):
        cid = lax.axis_index("c")
        sid = lax.axis_index("s")
        lane = lax.iota(i32, 16)
        zeros16i = jnp.zeros((16,), i32)
        ones16i = jnp.ones((16,), i32)
        lane_off = lane * NBINS
        chunk_base = sid * CHUNK

        for bl in range(4):
            gb = cid * 4 + bl
            pltpu.sync_copy(
                nms_hbm.at[gb, pl.ds(chunk_base, CHUNK)], chunk_v.at[bl]
            )

            def zero_body(j, _):
                hist2d[pl.ds(j * 16, 16)] = zeros16i
                return 0

            lax.fori_loop(0, 16 * NBINS // 16, zero_body, 0, unroll=4)

            def hist_body(j, _):
                v = chunk_v[bl, pl.ds(j * 16, 16)]
                b1 = (v * NB).astype(i32)
                b2 = (v * 32768.0).astype(i32) - FSUB
                bins = jnp.where(b1 >= FOFF, b2, b1)
                plsc.addupdate_scatter(hist2d, [lane_off + bins], ones16i)
                return 0

            lax.fori_loop(0, NVEC, hist_body, 0, unroll=4)

            def lred_body(j, _):
                acc = zeros16i
                for l in range(16):
                    acc = acc + hist2d[pl.ds(l * NBINS + j * 16, 16)]
                local_hist[pl.ds(j * 16, 16)] = acc
                return 0

            lax.fori_loop(0, NBINS // 16, lred_body, 0)
            pltpu.sync_copy(local_hist, hist_sh.at[bl, sid])

        plsc.subcore_barrier()

        @pl.when(sid < 4)
        def _():
            bl = sid
            for t in range(16):
                pltpu.sync_copy(
                    hist_sh.at[bl, t], hist2d.at[pl.ds(t * NBINS, NBINS)])

            def tred_body(j, _):
                acc = zeros16i
                for t in range(16):
                    acc = acc + hist2d[pl.ds(t * NBINS + j * 16, 16)]
                local_hist[pl.ds(j * 16, 16)] = acc
                return 0

            lax.fori_loop(0, NBINS // 16, tred_body, 0)

            def suf_body(jr, carry):
                j = NBINS // 16 - 1 - jr
                beta, csum = carry
                h = local_hist[pl.ds(j * 16, 16)]
                cs = plsc.cumsum(h)
                tot = jnp.max(cs)
                suffix = (csum + tot) - cs + h
                bins = j * 16 + lane
                cand = jnp.max(jnp.where(suffix >= K, bins, -1))
                return jnp.maximum(beta, cand), csum + tot

            beta, _ = lax.fori_loop(
                0, NBINS // 16, suf_body, (jnp.int32(-1), jnp.int32(0)))
            beta = jnp.maximum(beta, jnp.int32(NB // 4))
            tval = jnp.where(
                beta >= FOFF,
                (beta + FSUB).astype(f32) * (1.0 / 32768.0),
                beta.astype(f32) * (1.0 / NB),
            )
            thresh_v[...] = jnp.broadcast_to(tval, (16,))
            pltpu.sync_copy(thresh_v, thresh_sh.at[bl])

        plsc.subcore_barrier()

        for bl in range(4):
            for j in range(CCAP // 16):
                cscore_v[bl, pl.ds(j * 16, 16)] = jnp.full((16,), -1.0, f32)
                cidx_v[bl, pl.ds(j * 16, 16)] = zeros16i
        pltpu.sync_copy(thresh_sh, tread_v)
        for bl in range(4):
            tv = tread_v[bl, pl.ds(0, 16)]

            def comp_body(j, cntv):
                v = chunk_v[bl, pl.ds(j * 16, 16)]
                m = v >= tv
                cs = plsc.cumsum(m.astype(i32))
                pos = cntv + cs - 1
                mw = m & (pos < CCAP)
                plsc.store_scatter(cscore_v.at[bl], [pos], v, mask=mw)
                idxs = chunk_base + j * 16 + lane
                plsc.store_scatter(cidx_v.at[bl], [pos], idxs, mask=mw)
                return cntv + plsc.all_reduce_population_count(m)

            lax.fori_loop(0, NVEC, comp_body, jnp.zeros((16,), i32), unroll=4)
        for bl in range(4):
            for j in range(CCAP // 16):
                sl = pl.ds(j * 16, 16)
                cscore_v[bl, sl] = cscore_v[bl, sl] * jnp.float32(1.0)
                cidx_v[bl, sl] = cidx_v[bl, sl] + 0
            pltpu.sync_copy(cscore_v.at[bl], pool_s_sh.at[bl, sid])
            pltpu.sync_copy(cidx_v.at[bl], pool_i_sh.at[bl, sid])

        plsc.subcore_barrier()

        @pl.when(sid < 4)
        def _():
            bl = sid
            gb = cid * 4 + bl
            for t in range(16):
                pltpu.sync_copy(pool_s_sh.at[bl, t], pools_s.at[pl.ds(t * CCAP, CCAP)])
                pltpu.sync_copy(pool_i_sh.at[bl, t], pools_i.at[pl.ds(t * CCAP, CCAP)])
            for j in range(PCAP // 16):
                dense_s[pl.ds(j * 16, 16)] = jnp.full((16,), -1.0, f32)
                dense_i[pl.ds(j * 16, 16)] = zeros16i

            def dcomp_body(j, cntv):
                s = pools_s[pl.ds(j * 16, 16)]
                idx = pools_i[pl.ds(j * 16, 16)]
                m = s > -0.5
                cs = plsc.cumsum(m.astype(i32))
                pos = cntv + cs - 1
                mw = m & (pos < PCAP)
                plsc.store_scatter(dense_s, [pos], s, mask=mw)
                plsc.store_scatter(dense_i, [pos], idx, mask=mw)
                return cntv + plsc.all_reduce_population_count(m)

            cntv = lax.fori_loop(
                0, 16 * CCAP // 16, dcomp_body, jnp.zeros((16,), i32))
            cnt = jnp.max(cntv)
            nv = jnp.minimum((cnt + 15) // 16, PCAP // 16)
            big = jnp.full((16,), jnp.int32(2**30), i32)

            def rank_body(r, _):
                def scan_body(q, st):
                    bs, bi, bp = st
                    s = dense_s[pl.ds(q * 16, 16)]
                    idx = dense_i[pl.ds(q * 16, 16)]
                    p = q * 16 + lane
                    better = (s > bs) | ((s == bs) & (idx < bi))
                    return (
                        jnp.where(better, s, bs),
                        jnp.where(better, idx, bi),
                        jnp.where(better, p, bp),
                    )

                init = (jnp.full((16,), -2.0, f32), big, big)
                bs, bi, bp = lax.fori_loop(0, nv, scan_body, init)
                mval = jnp.max(bs)
                tie = bs == mval
                mi_ = jnp.min(jnp.where(tie, bi, big))
                mp = jnp.min(jnp.where(tie & (bi == mi_), bp, big))
                mp = jnp.minimum(mp, jnp.int32(PCAP - 1))
                lane0 = lane == 0
                plsc.store_scatter(
                    dense_s, [jnp.broadcast_to(mp, (16,))],
                    jnp.full((16,), -1.0, f32), mask=lane0)
                plsc.store_scatter(
                    out_s, [jnp.broadcast_to(r, (16,))],
                    jnp.broadcast_to(mval, (16,)), mask=lane0)
                plsc.store_scatter(
                    out_i, [jnp.broadcast_to(r, (16,))],
                    jnp.broadcast_to(mi_, (16,)), mask=lane0)
                return 0

            lax.fori_loop(0, K, rank_body, 0)

            for j in range(KPAD // 16):
                sl = pl.ds(j * 16, 16)
                s = out_s[sl]
                idx = jnp.where(s >= DET_THRESHOLD, out_i[sl] % HW, zeros16i)
                m = (s >= DET_THRESHOLD).astype(f32)
                out_s[sl] = s * m
                out_i[sl] = idx
                out_x[sl] = (idx % W).astype(f32)
                out_y[sl] = (idx // W).astype(f32)
            pltpu.sync_copy(out_s, s_hbm.at[gb])
            pltpu.sync_copy(out_i, i_hbm.at[gb])
            pltpu.sync_copy(out_x, x_hbm.at[gb])
            pltpu.sync_copy(out_y, y_hbm.at[gb])

    return topk_k(nms_flat)


NW = 32
GPT = 1280
GPT_REAL = 1250


def _sc_gather(reg_flat, addr, maskv):
    mesh = plsc.VectorSubcoreMesh(core_axis_name="c", subcore_axis_name="s")

    @functools.partial(
        pl.kernel,
        mesh=mesh,
        out_type=jax.ShapeDtypeStruct((NW, GPT), jnp.float32),
        scratch_types=[
            pltpu.VMEM((GPT,), jnp.int32),
            pltpu.VMEM((GPT,), jnp.float32),
            pltpu.VMEM((GPT,), jnp.float32),
            pltpu.SemaphoreType.DMA,
        ],
    )
    def gather_k(reg_hbm, addr_hbm, mask_hbm, out_hbm, idx_v, rows_v, mask_v, sem):
        wid = lax.axis_index("s") * 2 + lax.axis_index("c")
        pltpu.sync_copy(addr_hbm.at[wid], idx_v)
        pltpu.sync_copy(mask_hbm.at[wid], mask_v)
        pltpu.async_copy(reg_hbm.at[idx_v], rows_v, sem).wait()
        for j in range(GPT // 16):
            sl = pl.ds(j * 16, 16)
            rows_v[sl] = rows_v[sl] * mask_v[sl]
        pltpu.sync_copy(rows_v, out_hbm.at[wid])

    return gather_k(reg_flat, addr, maskv)


def kernel(pred_heatmap, pred_regression):
    nms = _nms(pred_heatmap)
    flat = nms.reshape(B, CHW)
    scores_m, idx_sp, xs_m, ys_m = _sc_topk(flat)
    scores_m = scores_m[:, :K]
    inds = idx_sp[:, :K]

    scores_f = scores_m.reshape(-1)
    mask = (scores_f >= DET_THRESHOLD).astype(jnp.float32)

    addr = (
        inds[:, :, None]
        + (jnp.arange(R, dtype=jnp.int32) * HW)[None, None, :]
        + (jnp.arange(B, dtype=jnp.int32) * (R * HW))[:, None, None]
    ).reshape(NW, GPT_REAL)
    addr = jnp.concatenate(
        [addr, jnp.zeros((NW, GPT - GPT_REAL), jnp.int32)], axis=1
    )
    maskv = jnp.broadcast_to(mask.reshape(B, K, 1), (B, K, R)).reshape(NW, GPT_REAL)
    maskv = jnp.concatenate(
        [maskv, jnp.zeros((NW, GPT - GPT_REAL), jnp.float32)], axis=1
    )
    pois_m = _sc_gather(pred_regression.reshape(-1), addr, maskv)
    pois_m = pois_m[:, :GPT_REAL].reshape(B * K, R)

    out = jnp.concatenate(
        [
            scores_f[:, None],
            pois_m,
            jnp.stack(
                [xs_m[:, :K].reshape(-1), ys_m[:, :K].reshape(-1)], axis=1
            ),
        ],
        axis=1,
    )
    return out


# speedup vs baseline: 3.6352x; 1.1866x over previous
import functools

import jax
import jax.numpy as jnp
from jax import lax
from jax.experimental import pallas as pl
from jax.experimental.pallas import tpu as pltpu
from jax.experimental.pallas import tpu_sc as plsc

B, C, H, W = 8, 3, 192, 640
R = 50
K = 100
DET_THRESHOLD = 0.25
HW = H * W
CHW = C * H * W


def _nms_body(x_ref, o_ref):
    x = x_ref[0, 0]
    ninf = jnp.float32(-jnp.inf)
    ninf_row = jnp.full((1, W), ninf, dtype=jnp.float32)
    up = jnp.concatenate([x[1:, :], ninf_row], axis=0)
    down = jnp.concatenate([ninf_row, x[:-1, :]], axis=0)
    m = jnp.maximum(jnp.maximum(x, up), down)
    ninf_col = jnp.full((H, 1), ninf, dtype=jnp.float32)
    left = jnp.concatenate([m[:, 1:], ninf_col], axis=1)
    right = jnp.concatenate([ninf_col, m[:, :-1]], axis=1)
    pooled = jnp.maximum(jnp.maximum(m, left), right)
    o_ref[0, 0] = jnp.where(pooled == x, x, jnp.float32(0.0))


def _nms(heat):
    return pl.pallas_call(
        _nms_body,
        grid=(B, C),
        in_specs=[pl.BlockSpec((1, 1, H, W), lambda b, c: (b, c, 0, 0))],
        out_specs=pl.BlockSpec((1, 1, H, W), lambda b, c: (b, c, 0, 0)),
        out_shape=jax.ShapeDtypeStruct((B, C, H, W), jnp.float32),
    )(heat)


NB = 512
NBINS = 1024
FOFF = 504
FSUB = 32256 - FOFF
CHUNK = CHW // 16
NVEC = CHUNK // 16
CCAP = 128
PCAP = 1024
KPAD = 128


def _sc_topk(nms_flat):
    mesh = plsc.VectorSubcoreMesh(core_axis_name="c", subcore_axis_name="s")
    i32 = jnp.int32
    f32 = jnp.float32

    @functools.partial(
        pl.kernel,
        mesh=mesh,
        compiler_params=pltpu.CompilerParams(needs_layout_passes=False),
        out_type=(
            jax.ShapeDtypeStruct((B, KPAD), f32),
            jax.ShapeDtypeStruct((B, KPAD), i32),
            jax.ShapeDtypeStruct((B, KPAD), f32),
            jax.ShapeDtypeStruct((B, KPAD), f32),
        ),
        scratch_types=[
            pltpu.VMEM((4, CHUNK), f32),
            pltpu.VMEM((16 * NBINS,), i32),
            pltpu.VMEM((NBINS,), i32),
            pltpu.VMEM((16,), f32),
            pltpu.VMEM((4, 16), f32),
            pltpu.VMEM((4, CCAP), f32),
            pltpu.VMEM((4, CCAP), i32),
            pltpu.VMEM((16 * CCAP,), f32),
            pltpu.VMEM((16 * CCAP,), i32),
            pltpu.VMEM((PCAP,), f32),
            pltpu.VMEM((PCAP,), i32),
            pltpu.VMEM((KPAD,), f32),
            pltpu.VMEM((KPAD,), i32),
            pltpu.VMEM((KPAD,), f32),
            pltpu.VMEM((KPAD,), f32),
            pltpu.VMEM_SHARED((4, 16, NBINS), i32),
            pltpu.VMEM_SHARED((4, 16), f32),
            pltpu.VMEM_SHARED((4, 16, CCAP), f32),
            pltpu.VMEM_SHARED((4, 16, CCAP), i32),
        ],
    )
    def topk_k(nms_hbm, s_hbm, i_hbm, x_hbm, y_hbm,
               chunk_v, hist2d, local_hist, thresh_v, tread_v, cscore_v, cidx_v,
               pools_s, pools_i, dense_s, dense_i, out_s, out_i, out_x, out_y,
               hist_sh, thresh_sh, pool_s_sh, pool_i_sh):
        cid = lax.axis_index("c")
        sid = lax.axis_index("s")
        lane = lax.iota(i32, 16)
        zeros16i = jnp.zeros((16,), i32)
        ones16i = jnp.ones((16,), i32)
        lane_off = lane * NBINS
        chunk_base = sid * CHUNK

        for bl in range(4):
            gb = cid * 4 + bl
            pltpu.sync_copy(
                nms_hbm.at[gb, pl.ds(chunk_base, CHUNK)], chunk_v.at[bl]
            )

            @plsc.parallel_loop(0, 16 * NBINS, 16, unroll=4)
            def _(i):
                hist2d[pl.ds(i, 16)] = zeros16i

            @plsc.parallel_loop(0, CHUNK, 16, unroll=4)
            def _(i):
                v = chunk_v[bl, pl.ds(i, 16)]
                b1 = (v * NB).astype(i32)
                b2 = (v * 32768.0).astype(i32) - FSUB
                bins = jnp.where(b1 >= FOFF, b2, b1)
                plsc.addupdate_scatter(hist2d, [lane_off + bins], ones16i)

            @plsc.parallel_loop(0, NBINS, 16, unroll=2)
            def _(i):
                acc = zeros16i
                for l in range(16):
                    acc = acc + hist2d[pl.ds(l * NBINS + i, 16)]
                local_hist[pl.ds(i, 16)] = acc

            pltpu.sync_copy(local_hist, hist_sh.at[bl, sid])

        plsc.subcore_barrier()

        @pl.when(sid < 4)
        def _():
            bl = sid
            for t in range(16):
                pltpu.sync_copy(
                    hist_sh.at[bl, t], hist2d.at[pl.ds(t * NBINS, NBINS)])

            def tred_body(j, _):
                acc = zeros16i
                for t in range(16):
                    acc = acc + hist2d[pl.ds(t * NBINS + j * 16, 16)]
                local_hist[pl.ds(j * 16, 16)] = acc
                return 0

            lax.fori_loop(0, NBINS // 16, tred_body, 0)

            def suf_body(jr, carry):
                j = NBINS // 16 - 1 - jr
                beta, csum = carry
                h = local_hist[pl.ds(j * 16, 16)]
                cs = plsc.cumsum(h)
                tot = jnp.max(cs)
                suffix = (csum + tot) - cs + h
                bins = j * 16 + lane
                cand = jnp.max(jnp.where(suffix >= K, bins, -1))
                return jnp.maximum(beta, cand), csum + tot

            beta, _ = lax.fori_loop(
                0, NBINS // 16, suf_body, (jnp.int32(-1), jnp.int32(0)))
            beta = jnp.maximum(beta, jnp.int32(NB // 4))
            tval = jnp.where(
                beta >= FOFF,
                (beta + FSUB).astype(f32) * (1.0 / 32768.0),
                beta.astype(f32) * (1.0 / NB),
            )
            thresh_v[...] = jnp.broadcast_to(tval, (16,))
            pltpu.sync_copy(thresh_v, thresh_sh.at[bl])

        plsc.subcore_barrier()

        for bl in range(4):
            for j in range(CCAP // 16):
                cscore_v[bl, pl.ds(j * 16, 16)] = jnp.full((16,), -1.0, f32)
                cidx_v[bl, pl.ds(j * 16, 16)] = zeros16i
        pltpu.sync_copy(thresh_sh, tread_v)
        for bl in range(4):
            tv = tread_v[bl, pl.ds(0, 16)]

            @plsc.parallel_loop(
                0, CHUNK, 16, unroll=4, carry=jnp.zeros((16,), i32))
            def _(i, cntv):
                v = chunk_v[bl, pl.ds(i, 16)]
                m = v >= tv
                cs = plsc.cumsum(m.astype(i32))
                pos = cntv + cs - 1
                mw = m & (pos < CCAP)
                plsc.store_scatter(cscore_v.at[bl], [pos], v, mask=mw)
                idxs = chunk_base + i + lane
                plsc.store_scatter(cidx_v.at[bl], [pos], idxs, mask=mw)
                return cntv + plsc.all_reduce_population_count(m)
        for bl in range(4):
            for j in range(CCAP // 16):
                sl = pl.ds(j * 16, 16)
                cscore_v[bl, sl] = cscore_v[bl, sl] * jnp.float32(1.0)
                cidx_v[bl, sl] = cidx_v[bl, sl] + 0
            pltpu.sync_copy(cscore_v.at[bl], pool_s_sh.at[bl, sid])
            pltpu.sync_copy(cidx_v.at[bl], pool_i_sh.at[bl, sid])

        plsc.subcore_barrier()

        @pl.when(sid < 4)
        def _():
            bl = sid
            gb = cid * 4 + bl
            for t in range(16):
                pltpu.sync_copy(pool_s_sh.at[bl, t], pools_s.at[pl.ds(t * CCAP, CCAP)])
                pltpu.sync_copy(pool_i_sh.at[bl, t], pools_i.at[pl.ds(t * CCAP, CCAP)])
            for j in range(PCAP // 16):
                dense_s[pl.ds(j * 16, 16)] = jnp.full((16,), -1.0, f32)
                dense_i[pl.ds(j * 16, 16)] = zeros16i

            @plsc.parallel_loop(
                0, 16 * CCAP, 16, unroll=4, carry=jnp.zeros((16,), i32))
            def dcomp(i, cntv):
                s = pools_s[pl.ds(i, 16)]
                idx = pools_i[pl.ds(i, 16)]
                m = s > -0.5
                cs = plsc.cumsum(m.astype(i32))
                pos = cntv + cs - 1
                mw = m & (pos < PCAP)
                plsc.store_scatter(dense_s, [pos], s, mask=mw)
                plsc.store_scatter(dense_i, [pos], idx, mask=mw)
                return cntv + plsc.all_reduce_population_count(m)

            cnt = jnp.max(dcomp)
            nv = jnp.minimum((cnt + 15) // 16, PCAP // 16)
            big = jnp.full((16,), jnp.int32(2**30), i32)

            def rank_body(r, _):
                def scan_body(q, st):
                    bs, bi, bp = st
                    s = dense_s[pl.ds(q * 16, 16)]
                    idx = dense_i[pl.ds(q * 16, 16)]
                    p = q * 16 + lane
                    better = (s > bs) | ((s == bs) & (idx < bi))
                    return (
                        jnp.where(better, s, bs),
                        jnp.where(better, idx, bi),
                        jnp.where(better, p, bp),
                    )

                init = (jnp.full((16,), -2.0, f32), big, big)
                bs, bi, bp = lax.fori_loop(0, nv, scan_body, init)
                mval = jnp.max(bs)
                tie = bs == mval
                mi_ = jnp.min(jnp.where(tie, bi, big))
                mp = jnp.min(jnp.where(tie & (bi == mi_), bp, big))
                mp = jnp.minimum(mp, jnp.int32(PCAP - 1))
                lane0 = lane == 0
                plsc.store_scatter(
                    dense_s, [jnp.broadcast_to(mp, (16,))],
                    jnp.full((16,), -1.0, f32), mask=lane0)
                plsc.store_scatter(
                    out_s, [jnp.broadcast_to(r, (16,))],
                    jnp.broadcast_to(mval, (16,)), mask=lane0)
                plsc.store_scatter(
                    out_i, [jnp.broadcast_to(r, (16,))],
                    jnp.broadcast_to(mi_, (16,)), mask=lane0)
                return 0

            lax.fori_loop(0, K, rank_body, 0)

            for j in range(KPAD // 16):
                sl = pl.ds(j * 16, 16)
                s = out_s[sl]
                idx = jnp.where(s >= DET_THRESHOLD, out_i[sl] % HW, zeros16i)
                m = (s >= DET_THRESHOLD).astype(f32)
                out_s[sl] = s * m
                out_i[sl] = idx
                out_x[sl] = (idx % W).astype(f32)
                out_y[sl] = (idx // W).astype(f32)
            pltpu.sync_copy(out_s, s_hbm.at[gb])
            pltpu.sync_copy(out_i, i_hbm.at[gb])
            pltpu.sync_copy(out_x, x_hbm.at[gb])
            pltpu.sync_copy(out_y, y_hbm.at[gb])

    return topk_k(nms_flat)


NW = 32
GPT = 1280
GPT_REAL = 1250


def _sc_gather(reg_flat, addr, maskv):
    mesh = plsc.VectorSubcoreMesh(core_axis_name="c", subcore_axis_name="s")

    @functools.partial(
        pl.kernel,
        mesh=mesh,
        out_type=jax.ShapeDtypeStruct((NW, GPT), jnp.float32),
        scratch_types=[
            pltpu.VMEM((GPT,), jnp.int32),
            pltpu.VMEM((GPT,), jnp.float32),
            pltpu.VMEM((GPT,), jnp.float32),
            pltpu.SemaphoreType.DMA,
        ],
    )
    def gather_k(reg_hbm, addr_hbm, mask_hbm, out_hbm, idx_v, rows_v, mask_v, sem):
        wid = lax.axis_index("s") * 2 + lax.axis_index("c")
        pltpu.sync_copy(addr_hbm.at[wid], idx_v)
        pltpu.sync_copy(mask_hbm.at[wid], mask_v)
        pltpu.async_copy(reg_hbm.at[idx_v], rows_v, sem).wait()
        for j in range(GPT // 16):
            sl = pl.ds(j * 16, 16)
            rows_v[sl] = rows_v[sl] * mask_v[sl]
        pltpu.sync_copy(rows_v, out_hbm.at[wid])

    return gather_k(reg_flat, addr, maskv)


def kernel(pred_heatmap, pred_regression):
    nms = _nms(pred_heatmap)
    flat = nms.reshape(B, CHW)
    scores_m, idx_sp, xs_m, ys_m = _sc_topk(flat)
    scores_m = scores_m[:, :K]
    inds = idx_sp[:, :K]

    scores_f = scores_m.reshape(-1)
    mask = (scores_f >= DET_THRESHOLD).astype(jnp.float32)

    addr = (
        inds[:, :, None]
        + (jnp.arange(R, dtype=jnp.int32) * HW)[None, None, :]
        + (jnp.arange(B, dtype=jnp.int32) * (R * HW))[:, None, None]
    ).reshape(NW, GPT_REAL)
    addr = jnp.concatenate(
        [addr, jnp.zeros((NW, GPT - GPT_REAL), jnp.int32)], axis=1
    )
    maskv = jnp.broadcast_to(mask.reshape(B, K, 1), (B, K, R)).reshape(NW, GPT_REAL)
    maskv = jnp.concatenate(
        [maskv, jnp.zeros((NW, GPT - GPT_REAL), jnp.float32)], axis=1
    )
    pois_m = _sc_gather(pred_regression.reshape(-1), addr, maskv)
    pois_m = pois_m[:, :GPT_REAL].reshape(B * K, R)

    out = jnp.concatenate(
        [
            scores_f[:, None],
            pois_m,
            jnp.stack(
                [xs_m[:, :K].reshape(-1), ys_m[:, :K].reshape(-1)], axis=1
            ),
        ],
        axis=1,
    )
    return out
